# Initial kernel scaffold; baseline (speedup 1.0000x reference)
import functools

import jax
import jax.numpy as jnp
from jax import lax
from jax.experimental import pallas as pl
from jax.experimental.pallas import tpu as pltpu
from jax.experimental.pallas import tpu_sc as plsc

N, E, D, H, B, MD, OUT = 10000, 320000, 128, 128, 64, 30, 64
L = 16
NC, NS = 2, 16
NW = NC * NS
CHUNK = 128
CPT = 79
EPAD = NW * CPT * CHUNK
RPT = N // NS


def _build_sc_spmm(width):
    mesh = plsc.VectorSubcoreMesh(core_axis_name="c", subcore_axis_name="s")

    @functools.partial(
        pl.kernel,
        out_type=jax.ShapeDtypeStruct((NC * N, width), jnp.float32),
        mesh=mesh,
        scratch_types=[
            pltpu.VMEM((CPT, CHUNK), jnp.int32),
            pltpu.VMEM((CPT, CHUNK), jnp.int32),
            pltpu.VMEM((CPT, CHUNK), jnp.float32),
            pltpu.VMEM((CHUNK, width), jnp.float32),
            pltpu.VMEM_SHARED((N, width), jnp.float32),
            pltpu.SemaphoreType.DMA,
        ],
    )
    def spmm(row_hbm, col_hbm, ew_hbm, tab_hbm, z_hbm, out_hbm,
             row_v, col_v, ew_v, gbuf, acc_sh, sem):
        c = lax.axis_index("c")
        s = lax.axis_index("s")
        wid = c * NS + s
        rbase = s * RPT
        pltpu.sync_copy(z_hbm.at[pl.ds(rbase, RPT)],
                        acc_sh.at[pl.ds(rbase, RPT)])
        ebase = wid * CPT
        pltpu.sync_copy(row_hbm.at[pl.ds(ebase, CPT)], row_v)
        pltpu.sync_copy(col_hbm.at[pl.ds(ebase, CPT)], col_v)
        pltpu.sync_copy(ew_hbm.at[pl.ds(ebase, CPT)], ew_v)
        plsc.subcore_barrier()

        @pl.loop(0, CPT)
        def _(ci):
            pltpu.async_copy(tab_hbm.at[row_v.at[ci]], gbuf, sem).wait()

            @pl.loop(0, CHUNK)
            def _(e):
                sv = jnp.full((L,), ew_v[ci, e], jnp.float32)
                for j in range(width // L):
                    sl = pl.ds(j * L, L)
                    gbuf[e, sl] = gbuf[e, sl] * sv

            pltpu.sync_copy(gbuf, acc_sh.at[col_v.at[ci]], add=True)

        plsc.subcore_barrier()
        pltpu.sync_copy(acc_sh.at[pl.ds(rbase, RPT)],
                        out_hbm.at[pl.ds(c * N + rbase, RPT)])

    return spmm


_SC_DEG = _build_sc_spmm(16)
_SC_CONV = _build_sc_spmm(D)


def _tc_prep_body(degp_ref, x_ref, w1_ref, h1s_ref, dinv_ref):
    deg = degp_ref[0:N, 0:1] + degp_ref[N:2 * N, 0:1] + 1.0
    dinv = lax.rsqrt(deg)
    h1 = jnp.dot(x_ref[...], w1_ref[...], preferred_element_type=jnp.float32)
    h1s_ref[...] = dinv * h1
    dinv_ref[...] = dinv


_TC_PREP = pl.pallas_call(
    _tc_prep_body,
    out_shape=[jax.ShapeDtypeStruct((N, D), jnp.float32),
               jax.ShapeDtypeStruct((N, 1), jnp.float32)],
)


def _tc_mid_body(acc_ref, h1s_ref, dinv_ref, b1_ref, w2_ref, h2s_ref):
    dinv = dinv_ref[...]
    pre = dinv * (acc_ref[0:N] + acc_ref[N:2 * N] + h1s_ref[...]) + b1_ref[...]
    out1 = jnp.maximum(pre, 0.0)
    h2 = jnp.dot(out1, w2_ref[...], preferred_element_type=jnp.float32)
    h2s_ref[...] = dinv * h2


_TC_MID = pl.pallas_call(
    _tc_mid_body,
    out_shape=jax.ShapeDtypeStruct((N, D), jnp.float32),
)


def _tc_final_body(acc_ref, h2s_ref, dinv_ref, b2_ref, batch_ref, md_ref,
                   wm_ref, bm_ref, wfc_ref, bfc_ref, out_ref):
    dinv = dinv_ref[...]
    pre = dinv * (acc_ref[0:N] + acc_ref[N:2 * N] + h2s_ref[...]) + b2_ref[...]
    h2f = jnp.maximum(pre, 0.0)
    seg = lax.broadcasted_iota(jnp.int32, (B, N), 0)
    onehot = (batch_ref[...] == seg).astype(jnp.float32)
    sums = jnp.dot(onehot, h2f, preferred_element_type=jnp.float32)
    counts = jnp.sum(onehot, axis=1, keepdims=True)
    pooled = sums / jnp.maximum(counts, 1.0)
    rr = lax.broadcasted_iota(jnp.float32, (B, B), 0)
    cc = lax.broadcasted_iota(jnp.float32, (B, B), 1)
    ls = (cc < rr).astype(jnp.float32)
    fi = jnp.dot(ls, counts, preferred_element_type=jnp.float32)
    fi = fi - 64.0 * jnp.floor(fi / 64.0)
    sel = (cc == fi).astype(jnp.float32)
    md0 = jnp.dot(sel, md_ref[...], preferred_element_type=jnp.float32)
    mdh = jnp.maximum(
        jnp.dot(md0, wm_ref[...], preferred_element_type=jnp.float32)
        + bm_ref[...], 0.0)
    wfc = wfc_ref[...]
    out_ref[...] = (jnp.dot(pooled, wfc[0:H], preferred_element_type=jnp.float32)
                    + jnp.dot(mdh, wfc[H:2 * H],
                              preferred_element_type=jnp.float32)
                    + bfc_ref[...])


_TC_FINAL = pl.pallas_call(
    _tc_final_body,
    out_shape=jax.ShapeDtypeStruct((B, OUT), jnp.float32),
)


def kernel(x, edge_index, edge_attr, batch, metadata,
           W1, b1, W2, b2, Wm, bm, Wfc, bfc):
    pad = EPAD - E
    row2d = jnp.pad(edge_index[0], (0, pad)).reshape(NW * CPT, CHUNK)
    col2d = jnp.pad(edge_index[1], (0, pad)).reshape(NW * CPT, CHUNK)
    ew2d = jnp.pad(edge_attr, (0, pad)).reshape(NW * CPT, CHUNK)
    zeros16 = jnp.zeros((N, 16), jnp.float32)
    ones16 = jnp.ones((N, 16), jnp.float32)
    zerosD = jnp.zeros((N, D), jnp.float32)

    degp = _SC_DEG(row2d, col2d, ew2d, ones16, zeros16)
    h1s, dinv = _TC_PREP(degp, x, W1)
    acc1 = _SC_CONV(row2d, col2d, ew2d, h1s, zerosD)
    h2s = _TC_MID(acc1, h1s, dinv, b1.reshape(1, H), W2)
    acc2 = _SC_CONV(row2d, col2d, ew2d, h2s, zerosD)
    out = _TC_FINAL(acc2, h2s, dinv, b2.reshape(1, H), batch.reshape(1, N),
                    metadata, Wm, bm.reshape(1, H), Wfc, bfc.reshape(1, OUT))
    return out


# speedup vs baseline: 7.7175x; 7.7175x over previous
import functools

import jax
import jax.numpy as jnp
from jax import lax
from jax.experimental import pallas as pl
from jax.experimental.pallas import tpu as pltpu
from jax.experimental.pallas import tpu_sc as plsc

N, E, D, H, B, MD, OUT = 10000, 320000, 128, 128, 64, 30, 64
L = 16
NC, NS = 2, 16
NW = NC * NS
CHUNK = 128
CPT = 80
EPAD = NW * CPT * CHUNK
NP_ = 10240
RPT = NP_ // NS


def _build_sc_spmm(width):
    mesh = plsc.VectorSubcoreMesh(core_axis_name="c", subcore_axis_name="s")

    @functools.partial(
        pl.kernel,
        out_type=jax.ShapeDtypeStruct((NC * NP_, width), jnp.float32),
        mesh=mesh,
        scratch_types=[
            pltpu.VMEM((CPT, CHUNK), jnp.int32),
            pltpu.VMEM((CPT, CHUNK), jnp.int32),
            pltpu.VMEM((CPT, CHUNK), jnp.float32),
            pltpu.VMEM((CHUNK, width), jnp.float32),
            pltpu.VMEM_SHARED((NP_, width), jnp.float32),
            pltpu.SemaphoreType.DMA,
        ],
    )
    def spmm(row_hbm, col_hbm, ew_hbm, tab_hbm, z_hbm, out_hbm,
             row_v, col_v, ew_v, gbuf, acc_sh, sem):
        c = lax.axis_index("c")
        s = lax.axis_index("s")
        wid = c * NS + s
        rbase = s * RPT
        pltpu.sync_copy(z_hbm.at[pl.ds(rbase, RPT)],
                        acc_sh.at[pl.ds(rbase, RPT)])
        ebase = wid * CPT
        pltpu.sync_copy(row_hbm.at[pl.ds(ebase, CPT)], row_v)
        pltpu.sync_copy(col_hbm.at[pl.ds(ebase, CPT)], col_v)
        pltpu.sync_copy(ew_hbm.at[pl.ds(ebase, CPT)], ew_v)
        plsc.subcore_barrier()

        @pl.loop(0, CPT)
        def _(ci):
            pltpu.async_copy(tab_hbm.at[row_v.at[ci]], gbuf, sem).wait()

            @pl.loop(0, CHUNK, step=L)
            def _(e0):
                wv = ew_v[ci, pl.ds(e0, L)]
                for k in range(L):
                    sv = jnp.full((L,), wv[k], jnp.float32)
                    for j in range(width // L):
                        sl = pl.ds(j * L, L)
                        gbuf[e0 + k, sl] = gbuf[e0 + k, sl] * sv

            pltpu.sync_copy(gbuf, acc_sh.at[col_v.at[ci]], add=True)

        plsc.subcore_barrier()
        pltpu.sync_copy(acc_sh.at[pl.ds(rbase, RPT)],
                        out_hbm.at[pl.ds(c * NP_ + rbase, RPT)])

    return spmm


_SC_CONV = _build_sc_spmm(D)


def _build_sc_deg():
    mesh = plsc.VectorSubcoreMesh(core_axis_name="c", subcore_axis_name="s")

    @functools.partial(
        pl.kernel,
        out_type=jax.ShapeDtypeStruct((NC * NP_, D), jnp.float32),
        mesh=mesh,
        scratch_types=[
            pltpu.VMEM((CPT, CHUNK), jnp.int32),
            pltpu.VMEM((CPT, CHUNK), jnp.float32),
            pltpu.VMEM((CHUNK, D), jnp.float32),
            pltpu.VMEM_SHARED((NP_, D), jnp.float32),
        ],
    )
    def deg(col_hbm, ew_hbm, z_hbm, out_hbm, col_v, ew_v, wbuf, acc_sh):
        c = lax.axis_index("c")
        s = lax.axis_index("s")
        wid = c * NS + s
        rbase = s * RPT
        pltpu.sync_copy(z_hbm.at[pl.ds(rbase, RPT)],
                        acc_sh.at[pl.ds(rbase, RPT)])
        ebase = wid * CPT
        pltpu.sync_copy(col_hbm.at[pl.ds(ebase, CPT)], col_v)
        pltpu.sync_copy(ew_hbm.at[pl.ds(ebase, CPT)], ew_v)
        @pl.loop(0, CHUNK)
        def _(e):
            for j in range(D // L):
                wbuf[e, pl.ds(j * L, L)] = jnp.zeros((L,), jnp.float32)

        plsc.subcore_barrier()

        @pl.loop(0, CPT)
        def _(ci):
            @pl.loop(0, CHUNK, step=L)
            def _(e0):
                wv = ew_v[ci, pl.ds(e0, L)]
                for k in range(L):
                    wbuf[e0 + k, pl.ds(0, L)] = jnp.full((L,), wv[k],
                                                         jnp.float32)

            pltpu.sync_copy(wbuf, acc_sh.at[col_v.at[ci]], add=True)

        plsc.subcore_barrier()
        pltpu.sync_copy(acc_sh.at[pl.ds(rbase, RPT)],
                        out_hbm.at[pl.ds(c * NP_ + rbase, RPT)])

    return deg


_SC_DEG = _build_sc_deg()


def _tc_prep_body(degp_ref, x_ref, w1_ref, h1s_ref, dinv_ref):
    deg = degp_ref[0:NP_, 0:1] + degp_ref[NP_:2 * NP_, 0:1] + 1.0
    dinv = lax.rsqrt(deg)
    h1 = jnp.dot(x_ref[...], w1_ref[...], preferred_element_type=jnp.float32)
    h1s_ref[...] = dinv * h1
    dinv_ref[...] = dinv


_TC_PREP = pl.pallas_call(
    _tc_prep_body,
    out_shape=[jax.ShapeDtypeStruct((NP_, D), jnp.float32),
               jax.ShapeDtypeStruct((NP_, 1), jnp.float32)],
)


def _tc_mid_body(acc_ref, h1s_ref, dinv_ref, b1_ref, w2_ref, h2s_ref):
    dinv = dinv_ref[...]
    pre = dinv * (acc_ref[0:NP_] + acc_ref[NP_:2 * NP_] + h1s_ref[...]) + b1_ref[...]
    out1 = jnp.maximum(pre, 0.0)
    h2 = jnp.dot(out1, w2_ref[...], preferred_element_type=jnp.float32)
    h2s_ref[...] = dinv * h2


_TC_MID = pl.pallas_call(
    _tc_mid_body,
    out_shape=jax.ShapeDtypeStruct((NP_, D), jnp.float32),
)


def _tc_final_body(acc_ref, h2s_ref, dinv_ref, b2_ref, batch_ref, md_ref,
                   wm_ref, bm_ref, wfc_ref, bfc_ref, out_ref):
    dinv = dinv_ref[...]
    dinv = dinv[0:N]
    pre = (dinv * (acc_ref[0:N] + acc_ref[NP_:NP_ + N] + h2s_ref[0:N])
           + b2_ref[...])
    h2f = jnp.maximum(pre, 0.0)
    seg = lax.broadcasted_iota(jnp.int32, (B, N), 0)
    onehot = (batch_ref[...] == seg).astype(jnp.float32)
    sums = jnp.dot(onehot, h2f, preferred_element_type=jnp.float32)
    counts = jnp.sum(onehot, axis=1, keepdims=True)
    pooled = sums / jnp.maximum(counts, 1.0)
    rr = lax.broadcasted_iota(jnp.int32, (B, B), 0)
    cc = lax.broadcasted_iota(jnp.int32, (B, B), 1)
    ls = (cc < rr).astype(jnp.float32)
    fi = jnp.dot(ls, counts, preferred_element_type=jnp.float32)
    fi = fi - 64.0 * jnp.floor(fi / 64.0)
    sel = (cc.astype(jnp.float32) == fi).astype(jnp.float32)
    md0 = jnp.dot(sel, md_ref[...], preferred_element_type=jnp.float32)
    mdh = jnp.maximum(
        jnp.dot(md0, wm_ref[...], preferred_element_type=jnp.float32)
        + bm_ref[...], 0.0)
    wfc = wfc_ref[...]
    out_ref[...] = (jnp.dot(pooled, wfc[0:H], preferred_element_type=jnp.float32)
                    + jnp.dot(mdh, wfc[H:2 * H],
                              preferred_element_type=jnp.float32)
                    + bfc_ref[...])


_TC_FINAL = pl.pallas_call(
    _tc_final_body,
    out_shape=jax.ShapeDtypeStruct((B, OUT), jnp.float32),
)


def kernel(x, edge_index, edge_attr, batch, metadata,
           W1, b1, W2, b2, Wm, bm, Wfc, bfc):
    pad = EPAD - E
    row2d = jnp.pad(edge_index[0], (0, pad)).reshape(NW * CPT, CHUNK)
    col2d = jnp.pad(edge_index[1], (0, pad)).reshape(NW * CPT, CHUNK)
    ew2d = jnp.pad(edge_attr, (0, pad)).reshape(NW * CPT, CHUNK)
    x_p = jnp.pad(x, ((0, NP_ - N), (0, 0)))
    zerosD = jnp.zeros((NP_, D), jnp.float32)

    degp = _SC_DEG(col2d, ew2d, zerosD)
    h1s, dinv = _TC_PREP(degp, x_p, W1)
    acc1 = _SC_CONV(row2d, col2d, ew2d, h1s, zerosD)
    h2s = _TC_MID(acc1, h1s, dinv, b1.reshape(1, H), W2)
    acc2 = _SC_CONV(row2d, col2d, ew2d, h2s, zerosD)
    out = _TC_FINAL(acc2, h2s, dinv, b2.reshape(1, H), batch.reshape(1, N),
                    metadata, Wm, bm.reshape(1, H), Wfc, bfc.reshape(1, OUT))
    return out


# speedup vs baseline: 8.4897x; 1.1001x over previous
import functools

import jax
import jax.numpy as jnp
from jax import lax
from jax.experimental import pallas as pl
from jax.experimental.pallas import tpu as pltpu
from jax.experimental.pallas import tpu_sc as plsc

N, E, D, H, B, MD, OUT = 10000, 320000, 128, 128, 64, 30, 64
L = 16
NC, NS = 2, 16
NW = NC * NS
CHUNK = 64
CPT = 160
LBK = 32
NLB = CPT // LBK
EPAD = NW * CPT * CHUNK
NP_ = 10240
RPT = NP_ // NS
NBUF = 4


def _build_sc_spmm(width):
    mesh = plsc.VectorSubcoreMesh(core_axis_name="c", subcore_axis_name="s")

    @functools.partial(
        pl.kernel,
        out_type=jax.ShapeDtypeStruct((NC * NP_, width), jnp.float32),
        mesh=mesh,
        scratch_types=(
            [pltpu.VMEM((LBK, CHUNK), jnp.int32),
             pltpu.VMEM((LBK, CHUNK), jnp.int32),
             pltpu.VMEM((LBK, CHUNK), jnp.float32)]
            + [pltpu.VMEM((CHUNK, width), jnp.float32)] * NBUF
            + [pltpu.VMEM_SHARED((NP_, width), jnp.float32)]
            + [pltpu.SemaphoreType.DMA] * (2 * NBUF)
        ),
    )
    def spmm(row_hbm, col_hbm, ew_hbm, tab_hbm, z_hbm, out_hbm,
             row_v, col_v, ew_v, *rest):
        gbufs = rest[0:NBUF]
        acc_sh = rest[NBUF]
        gsems = rest[NBUF + 1:2 * NBUF + 1]
        ssems = rest[2 * NBUF + 1:3 * NBUF + 1]
        c = lax.axis_index("c")
        s = lax.axis_index("s")
        wid = c * NS + s
        rbase = s * RPT
        pltpu.sync_copy(z_hbm.at[pl.ds(rbase, RPT)],
                        acc_sh.at[pl.ds(rbase, RPT)])
        plsc.subcore_barrier()

        def scale(gbuf, ci):
            @pl.loop(0, CHUNK, step=L)
            def _(e0):
                wv = ew_v[ci, pl.ds(e0, L)]
                for k in range(L):
                    sv = jnp.full((L,), wv[k], jnp.float32)
                    for j in range(width // L):
                        sl = pl.ds(j * L, L)
                        gbuf[e0 + k, sl] = gbuf[e0 + k, sl] * sv

        ebase = wid * CPT

        @pl.loop(0, NLB)
        def _(b):
            base = ebase + b * LBK
            pltpu.sync_copy(row_hbm.at[pl.ds(base, LBK)], row_v)
            pltpu.sync_copy(col_hbm.at[pl.ds(base, LBK)], col_v)
            pltpu.sync_copy(ew_hbm.at[pl.ds(base, LBK)], ew_v)

            for k in range(2):
                pltpu.async_copy(tab_hbm.at[row_v.at[k]], gbufs[k], gsems[k])

            @pl.loop(0, LBK, step=NBUF)
            def _(ci):
                for k in range(NBUF):
                    m = ci + k
                    gbuf = gbufs[k]
                    pltpu.make_async_copy(tab_hbm.at[row_v.at[m]], gbuf,
                                          gsems[k]).wait()
                    kn = (k + 2) % NBUF

                    @pl.when(m >= 2)
                    def _():
                        pltpu.make_async_copy(gbufs[kn],
                                              acc_sh.at[col_v.at[m - 2]],
                                              ssems[kn]).wait()

                    @pl.when(m + 2 < LBK)
                    def _():
                        pltpu.async_copy(tab_hbm.at[row_v.at[m + 2]],
                                         gbufs[kn], gsems[kn])

                    scale(gbuf, m)
                    pltpu.async_copy(gbuf, acc_sh.at[col_v.at[m]], ssems[k],
                                     add=True)

            for k in range(NBUF - 2, NBUF):
                pltpu.make_async_copy(gbufs[k],
                                      acc_sh.at[col_v.at[LBK - NBUF + k]],
                                      ssems[k]).wait()

        plsc.subcore_barrier()
        pltpu.sync_copy(acc_sh.at[pl.ds(rbase, RPT)],
                        out_hbm.at[pl.ds(c * NP_ + rbase, RPT)])

    return spmm


_SC_CONV = _build_sc_spmm(D)


def _build_sc_deg():
    mesh = plsc.VectorSubcoreMesh(core_axis_name="c", subcore_axis_name="s")

    @functools.partial(
        pl.kernel,
        out_type=jax.ShapeDtypeStruct((NC * NP_, D), jnp.float32),
        mesh=mesh,
        scratch_types=[
            pltpu.VMEM((CPT, CHUNK), jnp.int32),
            pltpu.VMEM((CPT, CHUNK), jnp.float32),
            pltpu.VMEM((CHUNK, D), jnp.float32),
            pltpu.VMEM_SHARED((NP_, D), jnp.float32),
        ],
    )
    def deg(col_hbm, ew_hbm, z_hbm, out_hbm, col_v, ew_v, wbuf, acc_sh):
        c = lax.axis_index("c")
        s = lax.axis_index("s")
        wid = c * NS + s
        rbase = s * RPT
        pltpu.sync_copy(z_hbm.at[pl.ds(rbase, RPT)],
                        acc_sh.at[pl.ds(rbase, RPT)])
        ebase = wid * CPT
        pltpu.sync_copy(col_hbm.at[pl.ds(ebase, CPT)], col_v)
        pltpu.sync_copy(ew_hbm.at[pl.ds(ebase, CPT)], ew_v)
        @pl.loop(0, CHUNK)
        def _(e):
            for j in range(D // L):
                wbuf[e, pl.ds(j * L, L)] = jnp.zeros((L,), jnp.float32)

        plsc.subcore_barrier()

        @pl.loop(0, CPT)
        def _(ci):
            @pl.loop(0, CHUNK, step=L)
            def _(e0):
                wv = ew_v[ci, pl.ds(e0, L)]
                for k in range(L):
                    wbuf[e0 + k, pl.ds(0, L)] = jnp.full((L,), wv[k],
                                                         jnp.float32)

            pltpu.sync_copy(wbuf, acc_sh.at[col_v.at[ci]], add=True)

        plsc.subcore_barrier()
        pltpu.sync_copy(acc_sh.at[pl.ds(rbase, RPT)],
                        out_hbm.at[pl.ds(c * NP_ + rbase, RPT)])

    return deg


_SC_DEG = _build_sc_deg()


def _tc_prep_body(degp_ref, x_ref, w1_ref, h1s_ref, dinv_ref):
    deg = degp_ref[0:NP_, 0:1] + degp_ref[NP_:2 * NP_, 0:1] + 1.0
    dinv = lax.rsqrt(deg)
    h1 = jnp.dot(x_ref[...], w1_ref[...], preferred_element_type=jnp.float32)
    h1s_ref[...] = dinv * h1
    dinv_ref[...] = dinv


_TC_PREP = pl.pallas_call(
    _tc_prep_body,
    out_shape=[jax.ShapeDtypeStruct((NP_, D), jnp.float32),
               jax.ShapeDtypeStruct((NP_, 1), jnp.float32)],
)


def _tc_mid_body(acc_ref, h1s_ref, dinv_ref, b1_ref, w2_ref, h2s_ref):
    dinv = dinv_ref[...]
    pre = dinv * (acc_ref[0:NP_] + acc_ref[NP_:2 * NP_] + h1s_ref[...]) + b1_ref[...]
    out1 = jnp.maximum(pre, 0.0)
    h2 = jnp.dot(out1, w2_ref[...], preferred_element_type=jnp.float32)
    h2s_ref[...] = dinv * h2


_TC_MID = pl.pallas_call(
    _tc_mid_body,
    out_shape=jax.ShapeDtypeStruct((NP_, D), jnp.float32),
)


def _tc_final_body(acc_ref, h2s_ref, dinv_ref, b2_ref, batch_ref, md_ref,
                   wm_ref, bm_ref, wfc_ref, bfc_ref, out_ref):
    dinv = dinv_ref[...]
    dinv = dinv[0:N]
    pre = (dinv * (acc_ref[0:N] + acc_ref[NP_:NP_ + N] + h2s_ref[0:N])
           + b2_ref[...])
    h2f = jnp.maximum(pre, 0.0)
    seg = lax.broadcasted_iota(jnp.int32, (B, N), 0)
    onehot = (batch_ref[...] == seg).astype(jnp.float32)
    sums = jnp.dot(onehot, h2f, preferred_element_type=jnp.float32)
    counts = jnp.sum(onehot, axis=1, keepdims=True)
    pooled = sums / jnp.maximum(counts, 1.0)
    rr = lax.broadcasted_iota(jnp.int32, (B, B), 0)
    cc = lax.broadcasted_iota(jnp.int32, (B, B), 1)
    ls = (cc < rr).astype(jnp.float32)
    fi = jnp.dot(ls, counts, preferred_element_type=jnp.float32)
    fi = fi - 64.0 * jnp.floor(fi / 64.0)
    sel = (cc.astype(jnp.float32) == fi).astype(jnp.float32)
    md0 = jnp.dot(sel, md_ref[...], preferred_element_type=jnp.float32)
    mdh = jnp.maximum(
        jnp.dot(md0, wm_ref[...], preferred_element_type=jnp.float32)
        + bm_ref[...], 0.0)
    wfc = wfc_ref[...]
    out_ref[...] = (jnp.dot(pooled, wfc[0:H], preferred_element_type=jnp.float32)
                    + jnp.dot(mdh, wfc[H:2 * H],
                              preferred_element_type=jnp.float32)
                    + bfc_ref[...])


_TC_FINAL = pl.pallas_call(
    _tc_final_body,
    out_shape=jax.ShapeDtypeStruct((B, OUT), jnp.float32),
)


def kernel(x, edge_index, edge_attr, batch, metadata,
           W1, b1, W2, b2, Wm, bm, Wfc, bfc):
    pad = EPAD - E
    row2d = jnp.pad(edge_index[0], (0, pad)).reshape(NW * CPT, CHUNK)
    col2d = jnp.pad(edge_index[1], (0, pad)).reshape(NW * CPT, CHUNK)
    ew2d = jnp.pad(edge_attr, (0, pad)).reshape(NW * CPT, CHUNK)
    x_p = jnp.pad(x, ((0, NP_ - N), (0, 0)))
    zerosD = jnp.zeros((NP_, D), jnp.float32)

    degp = _SC_DEG(col2d, ew2d, zerosD)
    h1s, dinv = _TC_PREP(degp, x_p, W1)
    acc1 = _SC_CONV(row2d, col2d, ew2d, h1s, zerosD)
    h2s = _TC_MID(acc1, h1s, dinv, b1.reshape(1, H), W2)
    acc2 = _SC_CONV(row2d, col2d, ew2d, h2s, zerosD)
    out = _TC_FINAL(acc2, h2s, dinv, b2.reshape(1, H), batch.reshape(1, N),
                    metadata, Wm, bm.reshape(1, H), Wfc, bfc.reshape(1, OUT))
    return out
